# baseline (speedup 1.0000x reference)
import jax
import jax.numpy as jnp
from jax.experimental import pallas as pl
from jax.experimental.pallas import tpu as pltpu

B = 256
MAX_N = 64
NVT = 20
HS = 501
NZ = 56
HP = 512
GP = 3 * HP
NP = 128
ZP = 128
PHASES = 8
PLEN = MAX_N // PHASES


def _body(oh_ref, adjP_ref, wih_ref, whh_ref, bhn_ref, wgm_ref, wgmb_ref,
          w1_ref, b1_ref, w2_ref, b2_ref,
          mu_ref, lv_ref, m_ref, hv_ref):
    m_ref[...] = jnp.zeros_like(m_ref)
    vlane = jax.lax.broadcasted_iota(jnp.int32, (1, MAX_N), 1)

    def gru(v, h_in):
        onehot = oh_ref[pl.ds(v, 1)][0]
        gi = jnp.dot(wih_ref[...], onehot,
                     preferred_element_type=jnp.float32)
        gh = jnp.dot(whh_ref[...], h_in.astype(jnp.bfloat16),
                     preferred_element_type=jnp.float32)
        rz = jax.nn.sigmoid(gi[0:2 * HP] + gh[0:2 * HP])
        r, z = rz[0:HP], rz[HP:2 * HP]
        n = jnp.tanh(gi[2 * HP:GP] + r * (gh[2 * HP:GP] + bhn_ref[...]))
        return n + z * (h_in - n)

    def msg(v, hv):
        vmask = (vlane == v).astype(jnp.float32)
        gmb = jnp.sum(wgmb_ref[...] * vmask, axis=1, keepdims=True)
        gm = jnp.dot(wgm_ref[...], hv.astype(jnp.bfloat16),
                     preferred_element_type=jnp.float32) + gmb
        return jax.nn.sigmoid(gm[0:HP]) * gm[HP:2 * HP]

    def make_pair(pref):
        def pair(i, carry):
            del carry
            v0 = 2 * i
            v1 = v0 + 1
            cv0 = adjP_ref[pl.ds(v0, 1)][0, 0:pref]
            cv1 = adjP_ref[pl.ds(v1, 1)][0, 0:pref]
            h0 = None
            b1v = None
            for k0 in range(0, pref, PLEN):
                blk = [m_ref[k0 + j] for j in range(PLEN)]
                sA = [blk[j] * cv0[k0 + j] for j in range(PLEN)]
                sB = [blk[j] * cv1[k0 + j] for j in range(PLEN)]
                tA = (((sA[0] + sA[1]) + (sA[2] + sA[3]))
                      + ((sA[4] + sA[5]) + (sA[6] + sA[7]))).astype(jnp.float32)
                tB = (((sB[0] + sB[1]) + (sB[2] + sB[3]))
                      + ((sB[4] + sB[5]) + (sB[6] + sB[7]))).astype(jnp.float32)
                h0 = tA if h0 is None else h0 + tA
                b1v = tB if b1v is None else b1v + tB
            hv0 = gru(v0, h0)
            mv0 = msg(v0, hv0)
            c01 = adjP_ref[pl.ds(v1, 1), pl.ds(v0, 1)].reshape(1, B)
            mv0b = mv0.astype(jnp.bfloat16)
            h1 = b1v + (mv0b * c01).astype(jnp.float32)
            hv1 = gru(v1, h1)
            mv1 = msg(v1, hv1)
            hv_ref[...] = hv1
            m_ref[pl.ds(v0, 1)] = mv0b[None]
            m_ref[pl.ds(v1, 1)] = mv1.astype(jnp.bfloat16)[None]
            return 0
        return pair

    for p in range(PHASES):
        jax.lax.fori_loop(p * PLEN // 2, (p + 1) * PLEN // 2,
                          make_pair((p + 1) * PLEN), 0)
    hv = hv_ref[...]
    mu_ref[...] = jnp.dot(w1_ref[...], hv,
                          preferred_element_type=jnp.float32) + b1_ref[...]
    lv_ref[...] = jnp.dot(w2_ref[...], hv,
                          preferred_element_type=jnp.float32) + b2_ref[...]


def _pack3(w, cols, dtype):
    out = jnp.zeros((GP, cols), jnp.float32)
    for k in range(3):
        out = out.at[k * HP:k * HP + HS, :w.shape[1]].set(
            w[k * HS:(k + 1) * HS, :])
    return out.astype(dtype)


def kernel(node_types, adj, gru_Wih, gru_Whh, gru_bih, gru_bhh,
           Wg, bg, Wm, W1, b1, W2, b2):
    f32, bf16 = jnp.float32, jnp.bfloat16
    oh = jax.nn.one_hot(node_types.T, NP, axis=1, dtype=bf16)
    adjP = jnp.transpose(adj, (2, 1, 0))[:, :, None, :].astype(bf16)

    bfold = gru_bih + jnp.concatenate(
        [gru_bhh[:HS], gru_bhh[HS:2 * HS], jnp.zeros((HS,), f32)])
    wih = _pack3(gru_Wih + bfold[:, None], NP, bf16)
    whh = _pack3(gru_Whh, HP, bf16)
    bhn = jnp.zeros((HP, 1), f32).at[:HS, 0].set(gru_bhh[2 * HS:])

    wgm = (jnp.zeros((2 * HP, HP), f32)
           .at[:HS, :HS].set(Wg[:, :HS])
           .at[HP:HP + HS, :HS].set(Wm[:, :HS])).astype(bf16)
    wgmb = (jnp.zeros((2 * HP, MAX_N), f32)
            .at[:HS, :].set(bg[:, None] + Wg[:, HS:])
            .at[HP:HP + HS, :].set(Wm[:, HS:]))

    w1 = jnp.zeros((ZP, HP), f32).at[:NZ, :HS].set(W1)
    b1p = jnp.zeros((ZP, 1), f32).at[:NZ, 0].set(b1)
    w2 = jnp.zeros((ZP, HP), f32).at[:NZ, :HS].set(W2)
    b2p = jnp.zeros((ZP, 1), f32).at[:NZ, 0].set(b2)

    mu, lv = pl.pallas_call(
        _body,
        out_shape=(jax.ShapeDtypeStruct((ZP, B), f32),
                   jax.ShapeDtypeStruct((ZP, B), f32)),
        scratch_shapes=[pltpu.VMEM((MAX_N, HP, B), bf16),
                        pltpu.VMEM((HP, B), f32)],
        compiler_params=pltpu.CompilerParams(
            vmem_limit_bytes=120 * 1024 * 1024),
    )(oh, adjP, wih, whh, bhn, wgm, wgmb, w1, b1p, w2, b2p)
    return (mu.T[:, :NZ], lv.T[:, :NZ])


# speedup vs baseline: 1.0528x; 1.0528x over previous
import jax
import jax.numpy as jnp
from jax.experimental import pallas as pl
from jax.experimental.pallas import tpu as pltpu

B = 256
MAX_N = 64
NVT = 20
HS = 501
NZ = 56
HP = 512
GP = 3 * HP
NP = 128
ZP = 128
PHASES = 8
PLEN = MAX_N // PHASES


def _body(oh_ref, adjP_ref, wih_ref, whh_ref, bhn_ref, wgm_ref, wgmb_ref,
          w1_ref, b1_ref, w2_ref, b2_ref,
          mu_ref, lv_ref, m_ref, hv_ref):
    m_ref[...] = jnp.zeros_like(m_ref)
    vlane = jax.lax.broadcasted_iota(jnp.int32, (1, MAX_N), 1)

    def gru(v, h_in):
        onehot = oh_ref[pl.ds(v, 1)][0]
        gi = jnp.dot(wih_ref[...], onehot,
                     preferred_element_type=jnp.float32)
        gh = jnp.dot(whh_ref[...], h_in.astype(jnp.bfloat16),
                     preferred_element_type=jnp.float32)
        rz = jax.nn.sigmoid(gi[0:2 * HP] + gh[0:2 * HP])
        r, z = rz[0:HP], rz[HP:2 * HP]
        n = jnp.tanh(gi[2 * HP:GP] + r * (gh[2 * HP:GP] + bhn_ref[...]))
        return n + z * (h_in - n)

    def msg(v, hv):
        vmask = (vlane == v).astype(jnp.float32)
        gmb = jnp.sum(wgmb_ref[...] * vmask, axis=1, keepdims=True)
        gm = jnp.dot(wgm_ref[...], hv.astype(jnp.bfloat16),
                     preferred_element_type=jnp.float32) + gmb
        return jax.nn.sigmoid(gm[0:HP]) * gm[HP:2 * HP]

    def make_step(pref):
        def step(v, carry):
            del carry
            cv = adjP_ref[pl.ds(v, 1)][0, 0:pref]
            h_in = None
            for k0 in range(0, pref, PLEN):
                s = [m_ref[k0 + j] * cv[k0 + j] for j in range(PLEN)]
                t8 = (((s[0] + s[1]) + (s[2] + s[3]))
                      + ((s[4] + s[5]) + (s[6] + s[7])))
                t8 = t8.astype(jnp.float32)
                h_in = t8 if h_in is None else h_in + t8
            hv = gru(v, h_in)
            hv_ref[...] = hv
            mv = msg(v, hv)
            m_ref[pl.ds(v, 1)] = mv.astype(jnp.bfloat16)[None]
            return 0
        return step

    for p in range(PHASES):
        jax.lax.fori_loop(p * PLEN, (p + 1) * PLEN,
                          make_step((p + 1) * PLEN), 0)
    hv = hv_ref[...]
    mu_ref[...] = jnp.dot(w1_ref[...], hv,
                          preferred_element_type=jnp.float32) + b1_ref[...]
    lv_ref[...] = jnp.dot(w2_ref[...], hv,
                          preferred_element_type=jnp.float32) + b2_ref[...]


def _pack3(w, cols, dtype):
    out = jnp.zeros((GP, cols), jnp.float32)
    for k in range(3):
        out = out.at[k * HP:k * HP + HS, :w.shape[1]].set(
            w[k * HS:(k + 1) * HS, :])
    return out.astype(dtype)


def kernel(node_types, adj, gru_Wih, gru_Whh, gru_bih, gru_bhh,
           Wg, bg, Wm, W1, b1, W2, b2):
    f32, bf16 = jnp.float32, jnp.bfloat16
    oh = jax.nn.one_hot(node_types.T, NP, axis=1, dtype=bf16)
    adjP = jnp.transpose(adj, (2, 1, 0))[:, :, None, :].astype(bf16)

    bfold = gru_bih + jnp.concatenate(
        [gru_bhh[:HS], gru_bhh[HS:2 * HS], jnp.zeros((HS,), f32)])
    wih = _pack3(gru_Wih + bfold[:, None], NP, bf16)
    whh = _pack3(gru_Whh, HP, bf16)
    bhn = jnp.zeros((HP, 1), f32).at[:HS, 0].set(gru_bhh[2 * HS:])

    wgm = (jnp.zeros((2 * HP, HP), f32)
           .at[:HS, :HS].set(Wg[:, :HS])
           .at[HP:HP + HS, :HS].set(Wm[:, :HS])).astype(bf16)
    wgmb = (jnp.zeros((2 * HP, MAX_N), f32)
            .at[:HS, :].set(bg[:, None] + Wg[:, HS:])
            .at[HP:HP + HS, :].set(Wm[:, HS:]))

    w1 = jnp.zeros((ZP, HP), f32).at[:NZ, :HS].set(W1)
    b1p = jnp.zeros((ZP, 1), f32).at[:NZ, 0].set(b1)
    w2 = jnp.zeros((ZP, HP), f32).at[:NZ, :HS].set(W2)
    b2p = jnp.zeros((ZP, 1), f32).at[:NZ, 0].set(b2)

    mu, lv = pl.pallas_call(
        _body,
        out_shape=(jax.ShapeDtypeStruct((ZP, B), f32),
                   jax.ShapeDtypeStruct((ZP, B), f32)),
        scratch_shapes=[pltpu.VMEM((MAX_N, HP, B), bf16),
                        pltpu.VMEM((HP, B), f32)],
        compiler_params=pltpu.CompilerParams(
            vmem_limit_bytes=120 * 1024 * 1024),
    )(oh, adjP, wih, whh, bhn, wgm, wgmb, w1, b1p, w2, b2p)
    return (mu.T[:, :NZ], lv.T[:, :NZ])


# speedup vs baseline: 1.1209x; 1.0647x over previous
import jax
import jax.numpy as jnp
from jax.experimental import pallas as pl
from jax.experimental.pallas import tpu as pltpu

B = 256
MAX_N = 64
NVT = 20
HS = 501
NZ = 56
HP = 512
GP = 3 * HP
NP = 128
ZP = 128
PHASES = 8
PLEN = MAX_N // PHASES


def _body(oh_ref, adjP_ref, wih_ref, whh_ref, bhn_ref, wgm_ref, wgmb_ref,
          w1_ref, b1_ref, w2_ref, b2_ref,
          mu_ref, lv_ref, m_ref, hv_ref):
    m_ref[...] = jnp.zeros_like(m_ref)
    vlane = jax.lax.broadcasted_iota(jnp.int32, (1, MAX_N), 1)

    def gru(v, h_in):
        onehot = oh_ref[pl.ds(v, 1)][0]
        gi = jnp.dot(wih_ref[...], onehot,
                     preferred_element_type=jnp.float32)
        gh = jnp.dot(whh_ref[...], h_in.astype(jnp.bfloat16),
                     preferred_element_type=jnp.float32)
        rz = jax.nn.sigmoid(gi[0:2 * HP] + gh[0:2 * HP])
        r, z = rz[0:HP], rz[HP:2 * HP]
        n = jnp.tanh(gi[2 * HP:GP] + r * (gh[2 * HP:GP] + bhn_ref[...]))
        return n + z * (h_in - n)

    def msg(v, hv):
        vmask = (vlane == v).astype(jnp.float32)
        gmb = jnp.sum(wgmb_ref[...] * vmask, axis=1, keepdims=True)
        gm = jnp.dot(wgm_ref[...], hv.astype(jnp.bfloat16),
                     preferred_element_type=jnp.float32) + gmb
        return jax.nn.sigmoid(gm[0:HP]) * gm[HP:2 * HP]

    def make_step(pref):
        def step(v, carry):
            del carry
            cv = adjP_ref[pl.ds(v, 1)][0, 0:pref]
            h_in = None
            for k0 in range(0, pref, PLEN):
                s = [m_ref[k0 + j] * cv[k0 + j] for j in range(PLEN)]
                t8 = (((s[0] + s[1]) + (s[2] + s[3]))
                      + ((s[4] + s[5]) + (s[6] + s[7])))
                t8 = t8.astype(jnp.float32)
                h_in = t8 if h_in is None else h_in + t8
            hv = gru(v, h_in)
            hv_ref[...] = hv
            mv = msg(v, hv)
            m_ref[pl.ds(v, 1)] = mv.astype(jnp.bfloat16)[None]
            return 0
        return step

    for p in range(PHASES):
        jax.lax.fori_loop(p * PLEN, (p + 1) * PLEN,
                          make_step((p + 1) * PLEN), 0)
    hv = hv_ref[...]
    mu_ref[...] = jnp.dot(w1_ref[...], hv,
                          preferred_element_type=jnp.float32) + b1_ref[...]
    lv_ref[...] = jnp.dot(w2_ref[...], hv,
                          preferred_element_type=jnp.float32) + b2_ref[...]


def _pad3(w3, cols, dtype):
    return jnp.pad(w3.astype(dtype),
                   ((0, 0), (0, HP - HS), (0, cols - w3.shape[2]))
                   ).reshape(3 * HP, cols)


def kernel(node_types, adj, gru_Wih, gru_Whh, gru_bih, gru_bhh,
           Wg, bg, Wm, W1, b1, W2, b2):
    f32, bf16 = jnp.float32, jnp.bfloat16
    oh = jax.nn.one_hot(node_types.T, NP, axis=1, dtype=bf16)
    adjP = jnp.transpose(adj, (2, 1, 0))[:, :, None, :].astype(bf16)

    bfold = gru_bih + jnp.concatenate(
        [gru_bhh[:2 * HS], jnp.zeros((HS,), f32)])
    wih = _pad3((gru_Wih + bfold[:, None]).reshape(3, HS, NVT), NP, bf16)
    whh = _pad3(gru_Whh.reshape(3, HS, HS), HP, bf16)
    bhn = jnp.pad(gru_bhh[2 * HS:, None], ((0, HP - HS), (0, 0)))

    wgm = jnp.pad(jnp.stack([Wg[:, :HS], Wm[:, :HS]]).astype(bf16),
                  ((0, 0), (0, HP - HS), (0, HP - HS))).reshape(2 * HP, HP)
    wgmb = jnp.pad(jnp.stack([bg[:, None] + Wg[:, HS:], Wm[:, HS:]]),
                   ((0, 0), (0, HP - HS), (0, 0))).reshape(2 * HP, MAX_N)

    w1 = jnp.pad(W1, ((0, ZP - NZ), (0, HP - HS)))
    b1p = jnp.pad(b1[:, None], ((0, ZP - NZ), (0, 0)))
    w2 = jnp.pad(W2, ((0, ZP - NZ), (0, HP - HS)))
    b2p = jnp.pad(b2[:, None], ((0, ZP - NZ), (0, 0)))

    mu, lv = pl.pallas_call(
        _body,
        out_shape=(jax.ShapeDtypeStruct((ZP, B), f32),
                   jax.ShapeDtypeStruct((ZP, B), f32)),
        scratch_shapes=[pltpu.VMEM((MAX_N, HP, B), bf16),
                        pltpu.VMEM((HP, B), f32)],
        compiler_params=pltpu.CompilerParams(
            vmem_limit_bytes=120 * 1024 * 1024),
    )(oh, adjP, wih, whh, bhn, wgm, wgmb, w1, b1p, w2, b2p)
    return (mu[:NZ].T, lv[:NZ].T)


# speedup vs baseline: 1.1246x; 1.0034x over previous
import jax
import jax.numpy as jnp
from jax.experimental import pallas as pl
from jax.experimental.pallas import tpu as pltpu

B = 256
MAX_N = 64
NVT = 20
HS = 501
NZ = 56
HP = 512
GP = 3 * HP
NP = 128
ZP = 128
PHASES = 8
PLEN = MAX_N // PHASES


def _body(oh_ref, adjP_ref, wih_ref, whh_ref, bhn_ref, wgm_ref, wgmb_ref,
          w1_ref, b1_ref, w2_ref, b2_ref,
          mu_ref, lv_ref, m_ref, hv_ref):
    m_ref[...] = jnp.zeros_like(m_ref)
    vlane = jax.lax.broadcasted_iota(jnp.int32, (1, MAX_N), 1)

    def gru(v, h_in):
        onehot = oh_ref[pl.ds(v, 1)][0]
        gi = jnp.dot(wih_ref[...], onehot,
                     preferred_element_type=jnp.float32)
        gh = jnp.dot(whh_ref[...], h_in.astype(jnp.bfloat16),
                     preferred_element_type=jnp.float32)
        rz = jax.nn.sigmoid(gi[0:2 * HP] + gh[0:2 * HP])
        r, z = rz[0:HP], rz[HP:2 * HP]
        n = jnp.tanh(gi[2 * HP:GP] + r * (gh[2 * HP:GP] + bhn_ref[...]))
        return n + z * (h_in - n)

    def msg(v, hv):
        vmask = (vlane == v).astype(jnp.float32)
        gmb = jnp.sum(wgmb_ref[...] * vmask, axis=1, keepdims=True)
        gm = jnp.dot(wgm_ref[...], hv.astype(jnp.bfloat16),
                     preferred_element_type=jnp.float32) + gmb
        return jax.nn.sigmoid(gm[0:HP]) * gm[HP:2 * HP]

    def make_step(pref):
        def step(v, carry):
            del carry
            cv = adjP_ref[pl.ds(v, 1)][0, 0:pref]
            h_in = None
            for k0 in range(0, pref, 2 * PLEN):
                n = min(2 * PLEN, pref - k0)
                s = [m_ref[k0 + j] * cv[k0 + j] for j in range(n)]
                while len(s) > 1:
                    s = [s[i] + s[i + 1] for i in range(0, len(s), 2)]
                t = s[0].astype(jnp.float32)
                h_in = t if h_in is None else h_in + t
            hv = gru(v, h_in)
            hv_ref[...] = hv
            mv = msg(v, hv)
            m_ref[pl.ds(v, 1)] = mv.astype(jnp.bfloat16)[None]
            return 0
        return step

    for p in range(PHASES):
        jax.lax.fori_loop(p * PLEN, (p + 1) * PLEN,
                          make_step((p + 1) * PLEN), 0)
    hv = hv_ref[...]
    mu_ref[...] = jnp.dot(w1_ref[...], hv,
                          preferred_element_type=jnp.float32) + b1_ref[...]
    lv_ref[...] = jnp.dot(w2_ref[...], hv,
                          preferred_element_type=jnp.float32) + b2_ref[...]


def _pad3(w3, cols, dtype):
    return jnp.pad(w3.astype(dtype),
                   ((0, 0), (0, HP - HS), (0, cols - w3.shape[2]))
                   ).reshape(3 * HP, cols)


def kernel(node_types, adj, gru_Wih, gru_Whh, gru_bih, gru_bhh,
           Wg, bg, Wm, W1, b1, W2, b2):
    f32, bf16 = jnp.float32, jnp.bfloat16
    oh = jax.nn.one_hot(node_types.T, NP, axis=1, dtype=bf16)
    adjP = jnp.transpose(adj.astype(bf16), (2, 1, 0))[:, :, None, :]

    bfold = gru_bih + jnp.concatenate(
        [gru_bhh[:2 * HS], jnp.zeros((HS,), f32)])
    wih = _pad3((gru_Wih + bfold[:, None]).reshape(3, HS, NVT), NP, bf16)
    whh = _pad3(gru_Whh.reshape(3, HS, HS), HP, bf16)
    bhn = jnp.pad(gru_bhh[2 * HS:, None], ((0, HP - HS), (0, 0)))

    wgm = jnp.pad(jnp.stack([Wg[:, :HS], Wm[:, :HS]]).astype(bf16),
                  ((0, 0), (0, HP - HS), (0, HP - HS))).reshape(2 * HP, HP)
    wgmb = jnp.pad(jnp.stack([bg[:, None] + Wg[:, HS:], Wm[:, HS:]]),
                   ((0, 0), (0, HP - HS), (0, 0))).reshape(2 * HP, MAX_N)

    w1 = jnp.pad(W1, ((0, ZP - NZ), (0, HP - HS)))
    b1p = jnp.pad(b1[:, None], ((0, ZP - NZ), (0, 0)))
    w2 = jnp.pad(W2, ((0, ZP - NZ), (0, HP - HS)))
    b2p = jnp.pad(b2[:, None], ((0, ZP - NZ), (0, 0)))

    mu, lv = pl.pallas_call(
        _body,
        out_shape=(jax.ShapeDtypeStruct((ZP, B), f32),
                   jax.ShapeDtypeStruct((ZP, B), f32)),
        scratch_shapes=[pltpu.VMEM((MAX_N, HP, B), bf16),
                        pltpu.VMEM((HP, B), f32)],
        compiler_params=pltpu.CompilerParams(
            vmem_limit_bytes=120 * 1024 * 1024),
    )(oh, adjP, wih, whh, bhn, wgm, wgmb, w1, b1p, w2, b2p)
    return (mu[:NZ].T, lv[:NZ].T)
